# baseline (speedup 1.0000x reference)
import functools

import jax
import jax.numpy as jnp
from jax import lax
from jax.experimental import pallas as pl
from jax.experimental.pallas import tpu as pltpu
from jax.experimental.pallas import tpu_sc as plsc

F = 26
B = 4096
D = 64
ROWS_PER_FIELD = 1000
N = B * F
NC = 2
NS = 16
NW = NC * NS
PER_W = N // NW
GSZ = 128
GPW = PER_W // GSZ
ORPC = GSZ * D // 128
NBUF = 4

_mesh = plsc.VectorSubcoreMesh(core_axis_name="c", subcore_axis_name="s")


@functools.partial(
    pl.kernel,
    out_type=jax.ShapeDtypeStruct((N * D // 128, 128), jnp.float32),
    mesh=_mesh,
    compiler_params=pltpu.CompilerParams(use_tc_tiling_on_sc=False),
    scratch_types=(
        [pltpu.VMEM((PER_W,), jnp.int32)]
        + [pltpu.VMEM((GSZ, D), jnp.float32)] * NBUF
        + [pltpu.SemaphoreType.DMA] * (2 * NBUF)
    ),
)
def _emb_lookup(x_hbm, table_hbm, out_hbm, idx_v, *bufs_sems):
    bufs = bufs_sems[:NBUF]
    gsems = bufs_sems[NBUF:2 * NBUF]
    wsems = bufs_sems[2 * NBUF:]
    wid = lax.axis_index("s") * NC + lax.axis_index("c")
    pltpu.sync_copy(x_hbm.at[pl.ds(wid * PER_W, PER_W)], idx_v)

    tview = table_hbm

    def add_offsets(g):
        def _add(i, c, g=g):
            s0 = i * 16
            s = lax.iota(jnp.int32, 16) + s0
            half = s0 // ORPC
            k = 2 * (s - half * ORPC) + half
            pos = k + g * GSZ
            sl = pl.ds(g * GSZ + s0, 16)
            idx_v[sl] = idx_v[sl] + (pos % F) * ROWS_PER_FIELD
            return c
        lax.fori_loop(0, GSZ // 16, _add, 0)

    def writeback(g, p):
        r0 = (wid * GPW + g) * ORPC
        for h in range(2):
            pltpu.async_copy(bufs[p].at[pl.ds(h * ORPC, ORPC), :],
                             out_hbm.at[pl.ds(r0, ORPC), pl.ds(h * D, D)],
                             wsems[p])

    def wait_writeback(g, p):
        r0 = (wid * GPW + g) * ORPC
        for h in range(2):
            pltpu.make_async_copy(bufs[p].at[pl.ds(h * ORPC, ORPC), :],
                                  out_hbm.at[pl.ds(r0, ORPC), pl.ds(h * D, D)],
                                  wsems[p]).wait()

    for g in range(NBUF):
        add_offsets(g)
        pltpu.async_copy(tview.at[idx_v.at[pl.ds(g * GSZ, GSZ)]], bufs[g], gsems[g])

    for g in range(GPW):
        p = g % NBUF
        j = g - 1 + NBUF
        if g >= 1 and j < GPW:
            q = (g - 1) % NBUF
            add_offsets(j)
            wait_writeback(g - 1, q)
            pltpu.async_copy(tview.at[idx_v.at[pl.ds(j * GSZ, GSZ)]], bufs[q], gsems[q])
        pltpu.make_async_copy(tview.at[idx_v.at[pl.ds(g * GSZ, GSZ)]], bufs[p],
                              gsems[p]).wait()
        writeback(g, p)

    for g in range(GPW - NBUF, GPW):
        wait_writeback(g, g % NBUF)


def kernel(x, table):
    xp = (x.astype(jnp.int32).reshape(N // GSZ, ORPC, 2)
          .transpose(0, 2, 1).reshape(N))
    out = _emb_lookup(xp, table)
    return out.reshape(B, F, D)
